# baseline (speedup 1.0000x reference)
import functools

import jax
import jax.numpy as jnp
from jax import lax
from jax.experimental import pallas as pl
from jax.experimental.pallas import tpu as pltpu
from jax.experimental.pallas import tpu_sc as plsc

N = 10000
E = 320000
D = 128

_NC = 2
_NS = 16
_NT = _NC * _NS
_CH = 32
_NCHUNK = 320
_IB = 16
_NIB = _NCHUNK // _IB
_NBUF = 4
_NROUND = _IB // _NBUF
_EPT = _CH * _NCHUNK
_EP = _EPT * _NT
_NPAD = 10240
_RPT = _NPAD // _NS


def _sc_body(table_h, src_h, dst_h, w_h, out_h,
             src_i, dst_i, w_i, g0_v, g1_v, g2_v, g3_v,
             s0_v, s1_v, s2_v, s3_v, agg_sh,
             gsem0, gsem1, gsem2, gsem3, ssem0, ssem1, ssem2, ssem3, isem):
    c = lax.axis_index("c")
    s = lax.axis_index("s")
    wid = c * _NS + s

    gbufs = (g0_v, g1_v, g2_v, g3_v)
    sbufs = (s0_v, s1_v, s2_v, s3_v)
    gsems = (gsem0, gsem1, gsem2, gsem3)
    ssems = (ssem0, ssem1, ssem2, ssem3)

    def _zrow(i, _):
        for q in range(8):
            z = jnp.zeros((16,), jnp.float32)
            for zb in sbufs:
                zb[i, pl.ds(q * 16, 16)] = z
        return 0
    lax.fori_loop(0, _CH, _zrow, 0)
    for r in range(_RPT // _CH):
        pltpu.async_copy(
            sbufs[r % _NBUF],
            agg_sh.at[pl.ds(s * _RPT + r * _CH, _CH)], isem)
    for r in range(_RPT // _CH):
        pltpu.make_async_copy(
            sbufs[r % _NBUF],
            agg_sh.at[pl.ds(s * _RPT + r * _CH, _CH)], isem).wait()

    pltpu.sync_copy(src_h.at[wid, 0], src_i.at[0])
    pltpu.sync_copy(dst_h.at[wid, 0], dst_i.at[0])
    pltpu.sync_copy(w_h.at[wid, 0], w_i.at[0])

    plsc.subcore_barrier()

    for b in range(_NBUF):
        pltpu.async_copy(table_h.at[src_i.at[0, b]], gbufs[b], gsems[b])

    def _block(m, _):
        p0 = m % 2

        def _round(k, _):
            @pl.when((k == 1) & (m + 1 < _NIB))
            def _():
                p = (m + 1) % 2
                pltpu.async_copy(src_h.at[wid, m + 1], src_i.at[p], isem)
                pltpu.async_copy(dst_h.at[wid, m + 1], dst_i.at[p], isem)
                pltpu.async_copy(w_h.at[wid, m + 1], w_i.at[p], isem)

            @pl.when((k == _NROUND - 1) & (m + 1 < _NIB))
            def _():
                p = (m + 1) % 2
                pltpu.make_async_copy(
                    src_h.at[wid, m + 1], src_i.at[p], isem).wait()
                pltpu.make_async_copy(
                    dst_h.at[wid, m + 1], dst_i.at[p], isem).wait()
                pltpu.make_async_copy(
                    w_h.at[wid, m + 1], w_i.at[p], isem).wait()

            for b in range(_NBUF):
                j = m * _IB + k * _NBUF + b
                loc = k * _NBUF + b
                gb, sb = gbufs[b], sbufs[b]
                pltpu.make_async_copy(
                    table_h.at[src_i.at[p0, loc]], gb, gsems[b]).wait()
                @pl.when(j >= _NBUF)
                def _():
                    pltpu.make_async_copy(
                        sb, agg_sh.at[dst_i.at[p0, loc]], ssems[b]).wait()
                def _group(g, _):
                    wv = w_i[p0, loc, pl.ds(g * 16, 16)]
                    for l in range(16):
                        wi = wv[l]
                        i = g * 16 + l
                        for q in range(8):
                            sl = pl.ds(q * 16, 16)
                            sb[i, sl] = gb[i, sl] * wi
                    return 0
                lax.fori_loop(0, _CH // 16, _group, 0)
                @pl.when(j + _NBUF < _NCHUNK)
                def _():
                    g2 = j + _NBUF
                    pltpu.async_copy(
                        table_h.at[src_i.at[(g2 // _IB) % 2, g2 % _IB]],
                        gb, gsems[b])
                pltpu.async_copy(
                    sb, agg_sh.at[dst_i.at[p0, loc]], ssems[b], add=True)
            return 0
        lax.fori_loop(0, _NROUND, _round, 0)
        return 0
    lax.fori_loop(0, _NIB, _block, 0)

    for b in range(_NBUF):
        pltpu.make_async_copy(
            sbufs[b], agg_sh.at[dst_i.at[0, 0]], ssems[b]).wait()

    plsc.subcore_barrier()

    pltpu.sync_copy(agg_sh.at[pl.ds(s * _RPT, _RPT)],
                    out_h.at[c, pl.ds(s * _RPT, _RPT)])


_sc_scatter = pl.kernel(
    _sc_body,
    out_type=jax.ShapeDtypeStruct((_NC, _NPAD, D), jnp.float32),
    mesh=plsc.VectorSubcoreMesh(core_axis_name="c", subcore_axis_name="s"),
    scratch_types=(
        [pltpu.VMEM((2, _IB, _CH), jnp.int32),
         pltpu.VMEM((2, _IB, _CH), jnp.int32),
         pltpu.VMEM((2, _IB, _CH), jnp.float32)]
        + [pltpu.VMEM((_CH, D), jnp.float32) for _ in range(2 * _NBUF)]
        + [pltpu.VMEM_SHARED((_NPAD, D), jnp.float32)]
        + [pltpu.SemaphoreType.DMA for _ in range(2 * _NBUF + 1)]
    ),
)


_ROWS = 2000


def _dot(a, b):
    return jax.lax.dot_general(
        a, b, (((1,), (0,)), ((), ())),
        precision=jax.lax.Precision.HIGHEST,
        preferred_element_type=jnp.float32)


def _root_body(x_ref, wrootT_ref, b_ref, o_ref):
    o_ref[...] = _dot(x_ref[...], wrootT_ref[...]) + b_ref[...]


def _dense1_body(part_ref, xr_ref, wrelT_ref, o_ref):
    agg = part_ref[0] + part_ref[1]
    h = _dot(agg, wrelT_ref[...]) + xr_ref[...]
    o_ref[...] = jnp.maximum(h, 0.0)


def _dense2_body(part_ref, xr_ref, wrelT_ref, wgT_ref, bg_ref, o_ref):
    agg = part_ref[0] + part_ref[1]
    h = _dot(agg, wrelT_ref[...]) + xr_ref[...]
    h = jnp.maximum(h, 0.0)
    out = _dot(h, wgT_ref[...])
    o_ref[...] = jnp.maximum(out + bg_ref[...], 0.0)


def _part_spec():
    return pl.BlockSpec((_NC, _ROWS, D), lambda i: (0, i, 0))


def _row_spec():
    return pl.BlockSpec((_ROWS, D), lambda i: (i, 0))


def _full_spec():
    return pl.BlockSpec((D, D), lambda i: (0, 0))


def _vec_spec():
    return pl.BlockSpec((1, D), lambda i: (0, 0))


def _root(x, wrootT, b):
    return pl.pallas_call(
        _root_body,
        grid=(N // _ROWS,),
        in_specs=[_row_spec(), _full_spec(), _vec_spec()],
        out_specs=_row_spec(),
        out_shape=jax.ShapeDtypeStruct((N, D), jnp.float32),
    )(x, wrootT, b.reshape(1, D))


def _dense1(part, xr, wrelT):
    return pl.pallas_call(
        _dense1_body,
        grid=(N // _ROWS,),
        in_specs=[_part_spec(), _row_spec(), _full_spec()],
        out_specs=_row_spec(),
        out_shape=jax.ShapeDtypeStruct((N, D), jnp.float32),
    )(part, xr, wrelT)


def _dense2(part, xr, wrelT, wgT, bg):
    return pl.pallas_call(
        _dense2_body,
        grid=(N // _ROWS,),
        in_specs=[_part_spec(), _row_spec(), _full_spec(), _full_spec(),
                  _vec_spec()],
        out_specs=_row_spec(),
        out_shape=jax.ShapeDtypeStruct((N, D), jnp.float32),
    )(part, xr, wrelT, wgT, bg.reshape(1, D))


def kernel(x, edge_index, edge_attributes, W_rel0, b_rel0, W_root0,
           W_rel1, b_rel1, W_root1, Wg, bg):
    src = edge_index[0].astype(jnp.int32)
    dst = edge_index[1].astype(jnp.int32)
    w = edge_attributes.astype(jnp.float32)

    pad = _EP - E
    spread = (jnp.arange(pad, dtype=jnp.int32) * 13) % N
    src_p = jnp.concatenate([src, spread])
    dst_p = jnp.concatenate([dst, spread])
    w_p = jnp.concatenate([w, jnp.zeros((pad,), jnp.float32)])
    src4 = src_p.reshape(_NT, _NIB, _IB, _CH)
    dst4 = dst_p.reshape(_NT, _NIB, _IB, _CH)
    w4 = w_p.reshape(_NT, _NIB, _IB, _CH)

    part0 = _sc_scatter(x, src4, dst4, w4)
    xr0 = _root(x, W_root0.T, b_rel0)
    h1 = _dense1(part0, xr0, W_rel0.T)
    part1 = _sc_scatter(h1, src4, dst4, w4)
    xr1 = _root(h1, W_root1.T, b_rel1)
    out = _dense2(part1, xr1, W_rel1.T, Wg.T, bg)
    return out
